# baseline (speedup 1.0000x reference)
import jax
import jax.numpy as jnp
import numpy as np
from jax import lax
from jax.experimental import pallas as pl

_B, _A, _F = 1024, 64, 128
_D, _FB, _CW = 6, 4, 32
_G = 32
_GA = _G * _A
_PW = 2 * _A

_DN0 = (((0,), (0,)), ((), ()))


def _tc_body(et_ref, atoms_ref, bt_ref, wa_ref, wb_ref, bsel_ref,
             dsel_ref, exp_ref, red_ref, iota_ref, eye_ref, out_ref):
    et = et_ref[...]
    atoms16 = atoms_ref[...].astype(jnp.bfloat16)
    bt16 = bt_ref[...].astype(jnp.bfloat16)

    valid16 = (et != -1).astype(jnp.bfloat16)
    ones_col = jnp.ones((_D, 1), jnp.bfloat16)
    deg = lax.dot_general(valid16, ones_col, _DN0,
                          preferred_element_type=jnp.float32)
    oh6 = (deg.astype(jnp.bfloat16) == dsel_ref[...]
           ).astype(jnp.bfloat16)[:, :_D]
    mask = lax.dot_general(oh6, exp_ref[...], (((1,), (0,)), ((), ())),
                           preferred_element_type=jnp.float32
                           ).astype(jnp.bfloat16)

    et16 = et.astype(jnp.bfloat16)
    iota_col = iota_ref[...]
    eye = eye_ref[...]

    sa_parts = []
    for p in range(_GA // _PW):
        sl = slice(p * _PW, (p + 1) * _PW)
        oh = [(et16[d:d + 1, sl] == iota_col).astype(jnp.bfloat16)
              for d in range(_D)]
        adjT = ((oh[0] + oh[1]) + (oh[2] + oh[3])) + ((oh[4] + oh[5]) + eye)
        sa_parts.append(lax.dot_general(adjT, atoms16[sl, :], _DN0,
                                        preferred_element_type=jnp.float32))
    sa16 = jnp.concatenate(sa_parts, axis=0).astype(jnp.bfloat16)

    y = jnp.dot(sa16, wa_ref[...], preferred_element_type=jnp.float32)
    y = y + lax.dot_general(bt16, wb_ref[...], _DN0,
                            preferred_element_type=jnp.float32)
    ym = y.astype(jnp.bfloat16) * mask
    out = jnp.dot(ym, red_ref[...], preferred_element_type=jnp.float32)
    out = out + lax.dot_general(oh6, bsel_ref[...],
                                (((1,), (0,)), ((), ())),
                                preferred_element_type=jnp.float32)
    out_ref[...] = out


def kernel(atoms, bonds, edges, W, b):
    atoms2 = atoms.reshape(_B * _A, _F)
    odd = (jnp.arange(_B, dtype=jnp.int32) & 1).reshape(_B, 1, 1)
    et = jnp.where(edges >= 0, edges + _A * odd, -1).reshape(_B * _A, _D).T
    bt = bonds.reshape(_B * _A, _D * _FB).T
    wa = W[:, :_F, :].transpose(1, 0, 2).reshape(_F, _D * _CW
                                                 ).astype(jnp.bfloat16)
    wb = jnp.tile(W[:, _F:, :].transpose(1, 0, 2).reshape(_FB, _D * _CW),
                  (_D, 1)).astype(jnp.bfloat16)
    bsel = b.astype(jnp.bfloat16)
    dsel = jnp.asarray(
        np.concatenate([np.arange(_D), np.full(122, 99.0)])
        .astype(np.float32)).reshape(1, 128).astype(jnp.bfloat16)
    exp = jnp.asarray(
        (np.arange(_D)[:, None] == np.arange(_D * _CW)[None, :] // _CW)
        .astype(np.float32)).astype(jnp.bfloat16)
    red = jnp.asarray(
        (np.arange(_D * _CW)[:, None] % _CW == np.arange(_CW)[None, :])
        .astype(np.float32)).astype(jnp.bfloat16)
    ii = np.arange(_PW)
    iota_c = jnp.asarray(np.broadcast_to(ii[:, None], (_PW, _PW))
                         .astype(np.float32)).astype(jnp.bfloat16)
    eye = jnp.asarray(np.eye(_PW, dtype=np.float32)).astype(jnp.bfloat16)

    out = pl.pallas_call(
        _tc_body,
        grid=(_B // _G,),
        in_specs=[
            pl.BlockSpec((_D, _GA), lambda i: (0, i)),
            pl.BlockSpec((_GA, _F), lambda i: (i, 0)),
            pl.BlockSpec((_D * _FB, _GA), lambda i: (0, i)),
            pl.BlockSpec((_F, _D * _CW), lambda i: (0, 0)),
            pl.BlockSpec((_D * _FB, _D * _CW), lambda i: (0, 0)),
            pl.BlockSpec((_D, _CW), lambda i: (0, 0)),
            pl.BlockSpec((1, 128), lambda i: (0, 0)),
            pl.BlockSpec((_D, _D * _CW), lambda i: (0, 0)),
            pl.BlockSpec((_D * _CW, _CW), lambda i: (0, 0)),
            pl.BlockSpec((_PW, _PW), lambda i: (0, 0)),
            pl.BlockSpec((_PW, _PW), lambda i: (0, 0)),
        ],
        out_specs=pl.BlockSpec((_GA, _CW), lambda i: (i, 0)),
        out_shape=jax.ShapeDtypeStruct((_B * _A, _CW), jnp.float32),
    )(et, atoms2, bt, wa, wb, bsel, dsel, exp, red, iota_c, eye)
    return out.reshape(_B, _A, _CW)
